# baseline (speedup 1.0000x reference)
import functools

import jax
import jax.numpy as jnp
from jax import lax
from jax.experimental import pallas as pl
from jax.experimental.pallas import tpu as pltpu
from jax.experimental.pallas import tpu_sc as plsc


def _sc_mesh_info():
    info = plsc.get_sparse_core_info()
    return info.num_cores, info.num_subcores


def _make_pool(B, Lc, Ld, D, LP):
    NC, NS = _sc_mesh_info()
    NW = NC * NS
    RW = B // NW
    CBC = 4
    CBD = 1
    NIDX = max(CBC * LP, CBD * Ld)
    NK = D // 16
    NB = 4

    mesh = plsc.VectorSubcoreMesh(core_axis_name="c", subcore_axis_name="s")

    @functools.partial(
        pl.kernel,
        out_type=jax.ShapeDtypeStruct((B // 8, 8, 2 * D), jnp.float32),
        mesh=mesh,
        scratch_types=(
            [pltpu.VMEM((RW * LP,), jnp.int32),
             pltpu.VMEM((RW * Ld,), jnp.int32)]
            + [pltpu.VMEM((NIDX, D), jnp.float32) for _ in range(NB)]
            + [pltpu.VMEM((CBC, D), jnp.float32) for _ in range(NB)]
            + [pltpu.SemaphoreType.DMA for _ in range(2 * NB + 1)]
        ),
        compiler_params=pltpu.CompilerParams(use_tc_tiling_on_sc=False),
    )
    def pool(ctx_hbm, doc_hbm, table_hbm, h_hbm, idxc_v, idxd_v, *bufs):
        rows = bufs[:NB]
        outs = bufs[NB:2 * NB]
        sems = bufs[2 * NB:3 * NB]
        semos = bufs[3 * NB:4 * NB]
        semi = bufs[4 * NB]
        wid = lax.axis_index("s") * NC + lax.axis_index("c")
        row0 = wid * RW

        pltpu.sync_copy(ctx_hbm.at[pl.ds(row0 * LP, RW * LP)], idxc_v)
        pltpu.async_copy(doc_hbm.at[pl.ds(row0 * Ld, RW * Ld)], idxd_v, semi)

        def one_pass(idx_v, L, LPP, CB, col0, U):
            scale = jnp.float32(1.0 / L)
            nch = RW // CB
            NR = CB * LPP

            def start(c, b):
                pltpu.async_copy(
                    table_hbm.at[idx_v.at[pl.ds(c * NR, NR)]],
                    rows[b].at[pl.ds(0, NR)], sems[b])

            def wait(b):
                pltpu.make_async_copy(
                    table_hbm.at[pl.ds(0, NR)],
                    rows[b].at[pl.ds(0, NR)], sems[b]).wait()

            def drain_out(b):
                pltpu.make_async_copy(
                    h_hbm.at[0, pl.ds(0, CB), pl.ds(0, D)],
                    outs[b].at[pl.ds(0, CB)], semos[b]).wait()

            def accum(c, b):
                rbuf = rows[b]
                outb = outs[b]
                for r in range(CB):
                    def jbody(j, accs, r=r):
                        base = r * LPP + U * j
                        acc = list(accs)
                        for u in range(U):
                            for k in range(NK):
                                acc[k] = acc[k] + rbuf[base + u,
                                                       pl.ds(16 * k, 16)]
                        return tuple(acc)
                    accs = lax.fori_loop(
                        0, L // U, jbody,
                        tuple(jnp.zeros((16,), jnp.float32)
                              for _ in range(NK)))
                    for k in range(NK):
                        outb[r, pl.ds(16 * k, 16)] = accs[k] * scale
                g = row0 // 8 + (c * CB) // 8
                s = (c * CB) % 8
                pltpu.async_copy(
                    outb.at[pl.ds(0, CB)],
                    h_hbm.at[g, pl.ds(s, CB), pl.ds(col0, D)],
                    semos[b])

            for i in range(NB - 1):
                start(i, i)

            def quad(t, carry):
                c0 = NB * t
                for i in range(NB):
                    c = c0 + i
                    wait(i)

                    @pl.when(c + (NB - 1) < nch)
                    def _(c=c, i=i):
                        start(c + (NB - 1), (i + NB - 1) % NB)

                    @pl.when(t > 0)
                    def _(i=i):
                        drain_out(i)

                    accum(c, i)
                return carry

            lax.fori_loop(0, nch // NB, quad, 0)
            for i in range(NB):
                drain_out(i)

        one_pass(idxc_v, Lc, LP, CBC, 0, 10)
        pltpu.make_async_copy(
            doc_hbm.at[pl.ds(0, RW * Ld)], idxd_v, semi).wait()
        one_pass(idxd_v, Ld, Ld, CBD, D, 25)

    return pool


def _mlp_body(h_ref, w_ref, b_ref, o_ref):
    hb = h_ref[...]
    G, S, D2 = hb.shape
    o_ref[...] = (
        jnp.dot(hb.reshape(G * S, D2), w_ref[...],
                preferred_element_type=jnp.float32)
        + b_ref[...])


def _mlp(h3, W, b2d):
    G8, _, D2 = h3.shape
    B = G8 * 8
    D = D2 // 2
    BB = 512
    return pl.pallas_call(
        _mlp_body,
        out_shape=jax.ShapeDtypeStruct((B, D), jnp.float32),
        grid=(B // BB,),
        in_specs=[
            pl.BlockSpec((BB // 8, 8, D2), lambda i: (i, 0, 0)),
            pl.BlockSpec((D2, D), lambda i: (0, 0)),
            pl.BlockSpec((1, D), lambda i: (0, 0)),
        ],
        out_specs=pl.BlockSpec((BB, D), lambda i: (i, 0)),
    )(h3, W, b2d)


def kernel(context, doc, word_embeds, W_mlp, b_mlp):
    B, Lc = context.shape
    _, Ld = doc.shape
    _, D = word_embeds.shape
    fctx = context.reshape(B * Lc)
    fdoc = doc.reshape(B * Ld)
    h3 = _make_pool(B, Lc, Ld, D, Lc)(fctx, fdoc, word_embeds)
    return _mlp(h3, W_mlp, b_mlp.reshape(1, D))


# speedup vs baseline: 1.0068x; 1.0068x over previous
import functools

import jax
import jax.numpy as jnp
from jax import lax
from jax.experimental import pallas as pl
from jax.experimental.pallas import tpu as pltpu
from jax.experimental.pallas import tpu_sc as plsc


def _sc_mesh_info():
    info = plsc.get_sparse_core_info()
    return info.num_cores, info.num_subcores


def _make_pool(B, Lc, Ld, D, LP):
    NC, NS = _sc_mesh_info()
    NW = NC * NS
    RW = B // NW
    CBC = 4
    CBD = 1
    NIDX = max(CBC * LP, CBD * Ld)
    NK = D // 16
    NB = 4

    mesh = plsc.VectorSubcoreMesh(core_axis_name="c", subcore_axis_name="s")

    @functools.partial(
        pl.kernel,
        out_type=jax.ShapeDtypeStruct((B // 8, 8, 2 * D), jnp.float32),
        mesh=mesh,
        scratch_types=(
            [pltpu.VMEM((RW * LP,), jnp.int32),
             pltpu.VMEM((RW * Ld,), jnp.int32)]
            + [pltpu.VMEM((NIDX, D), jnp.float32) for _ in range(NB)]
            + [pltpu.VMEM((CBC, D), jnp.float32) for _ in range(NB)]
            + [pltpu.SemaphoreType.DMA for _ in range(2 * NB + 1)]
        ),
        compiler_params=pltpu.CompilerParams(use_tc_tiling_on_sc=False),
    )
    def pool(ctx_hbm, doc_hbm, table_hbm, h_hbm, idxc_v, idxd_v, *bufs):
        rows = bufs[:NB]
        outs = bufs[NB:2 * NB]
        sems = bufs[2 * NB:3 * NB]
        semos = bufs[3 * NB:4 * NB]
        semi = bufs[4 * NB]
        wid = lax.axis_index("s") * NC + lax.axis_index("c")
        row0 = wid * RW

        pltpu.sync_copy(ctx_hbm.at[pl.ds(row0 * LP, RW * LP)], idxc_v)
        pltpu.async_copy(doc_hbm.at[pl.ds(row0 * Ld, RW * Ld)], idxd_v, semi)

        def one_pass(idx_v, L, LPP, CB, col0, U):
            scale = jnp.float32(1.0 / L)
            nch = RW // CB
            NR = CB * LPP

            def start(c, b):
                pltpu.async_copy(
                    table_hbm.at[idx_v.at[pl.ds(c * NR, NR)]],
                    rows[b].at[pl.ds(0, NR)], sems[b])

            def wait(b):
                pltpu.make_async_copy(
                    table_hbm.at[pl.ds(0, NR)],
                    rows[b].at[pl.ds(0, NR)], sems[b]).wait()

            def drain_out(b):
                pltpu.make_async_copy(
                    h_hbm.at[0, pl.ds(0, CB), pl.ds(0, D)],
                    outs[b].at[pl.ds(0, CB)], semos[b]).wait()

            def accum(c, b):
                rbuf = rows[b]
                outb = outs[b]
                for r in range(CB):
                    def jbody(j, accs, r=r):
                        base = r * LPP + U * j
                        acc = list(accs)
                        for u in range(U):
                            for k in range(NK):
                                acc[k] = acc[k] + rbuf[base + u,
                                                       pl.ds(16 * k, 16)]
                        return tuple(acc)
                    accs = lax.fori_loop(
                        0, L // U, jbody,
                        tuple(jnp.zeros((16,), jnp.float32)
                              for _ in range(NK)))
                    for k in range(NK):
                        outb[r, pl.ds(16 * k, 16)] = accs[k] * scale
                g = row0 // 8 + (c * CB) // 8
                s = (c * CB) % 8
                pltpu.async_copy(
                    outb.at[pl.ds(0, CB)],
                    h_hbm.at[g, pl.ds(s, CB), pl.ds(col0, D)],
                    semos[b])

            for i in range(NB - 1):
                start(i, i)

            def quad(t, carry):
                c0 = NB * t
                for i in range(NB):
                    c = c0 + i
                    wait(i)

                    @pl.when(c + (NB - 1) < nch)
                    def _(c=c, i=i):
                        start(c + (NB - 1), (i + NB - 1) % NB)

                    @pl.when(t > 0)
                    def _(i=i):
                        drain_out(i)

                    accum(c, i)
                return carry

            lax.fori_loop(0, nch // NB, quad, 0)
            for i in range(NB):
                drain_out(i)

        one_pass(idxc_v, Lc, LP, CBC, 0, 5)
        pltpu.make_async_copy(
            doc_hbm.at[pl.ds(0, RW * Ld)], idxd_v, semi).wait()
        one_pass(idxd_v, Ld, Ld, CBD, D, 8)

    return pool


def _mlp_body(h_ref, w_ref, b_ref, o_ref):
    hb = h_ref[...]
    G, S, D2 = hb.shape
    o_ref[...] = (
        jnp.dot(hb.reshape(G * S, D2), w_ref[...],
                preferred_element_type=jnp.float32)
        + b_ref[...])


def _mlp(h3, W, b2d):
    G8, _, D2 = h3.shape
    B = G8 * 8
    D = D2 // 2
    BB = 512
    return pl.pallas_call(
        _mlp_body,
        out_shape=jax.ShapeDtypeStruct((B, D), jnp.float32),
        grid=(B // BB,),
        in_specs=[
            pl.BlockSpec((BB // 8, 8, D2), lambda i: (i, 0, 0)),
            pl.BlockSpec((D2, D), lambda i: (0, 0)),
            pl.BlockSpec((1, D), lambda i: (0, 0)),
        ],
        out_specs=pl.BlockSpec((BB, D), lambda i: (i, 0)),
    )(h3, W, b2d)


def kernel(context, doc, word_embeds, W_mlp, b_mlp):
    B, Lc = context.shape
    _, Ld = doc.shape
    _, D = word_embeds.shape
    fctx = context.reshape(B * Lc)
    fdoc = doc.reshape(B * Ld)
    h3 = _make_pool(B, Lc, Ld, D, Lc)(fctx, fdoc, word_embeds)
    return _mlp(h3, W_mlp, b_mlp.reshape(1, D))
